# baseline (speedup 1.0000x reference)
import functools

import jax
import jax.numpy as jnp
from jax.experimental import pallas as pl
from jax.experimental.pallas import tpu as pltpu


def _pick_block(e: int) -> int:
    for b in (6400, 3200, 1600, 800, 640, 320, 160, 80, 40, 16, 8):
        if e % b == 0:
            return b
    return e


def _lattice_kernel(edge_emb_ref, rbf_ref, dvect_ref, idx_ref, batch_ref,
                    w1_ref, w2_ref, wrbf_ref, wout_ref,
                    out_ref, acc_ref, cum_ref, cums_ref,
                    *, num_graphs: int, nblocks: int, num_nodes: int):
    i = pl.program_id(0)
    f32 = jnp.float32

    @pl.when(i == 0)
    def _init():
        nodes = batch_ref[...]
        gio = jax.lax.broadcasted_iota(jnp.int32, (1, num_graphs), 1)
        cum = jnp.sum((nodes < gio).astype(jnp.int32), axis=0, keepdims=True)
        cum_ref[...] = cum.astype(jnp.int16)
        cums_ref[...] = jnp.concatenate(
            [cum[:, 1:], jnp.full((1, 1), num_nodes, jnp.int32)],
            axis=1).astype(jnp.int16)
        acc_ref[...] = jnp.zeros_like(acc_ref)

    bf16 = jnp.bfloat16
    x = edge_emb_ref[...]
    h = x @ w1_ref[...]
    h = h * (1.0 / (1.0 + jnp.exp(-h)))
    y = (h @ w2_ref[...]) * (rbf_ref[...] @ wrbf_ref[...])
    s = y @ wout_ref[...]
    s_row = jnp.transpose(s)

    d = dvect_ref[...]
    dx, dy, dz = d[0:1, :], d[1:2, :], d[2:3, :]
    n2 = dx * dx + dy * dy + dz * dz
    w = s_row / (jnp.sqrt(n2) + 1e-12)
    wpxx, wpyy, wpzz = w * (dx * dx), w * (dy * dy), w * (dz * dz)
    wpxy, wpxz, wpyz = w * (dx * dy), w * (dx * dz), w * (dy * dz)
    ones = jnp.ones_like(w)
    zeros6 = jnp.zeros((6,) + w.shape[1:], f32)
    contrib_t = jnp.concatenate(
        [wpxx, wpxy, wpxz, wpxy, wpyy, wpyz, wpxz, wpyz, wpzz, ones, zeros6],
        axis=0)

    idx = jnp.transpose(idx_ref[0]).astype(jnp.int16)
    c_lo = idx >= cum_ref[...]
    c_hi = idx >= cums_ref[...]
    oh = jnp.where(c_lo != c_hi, bf16(1.0), bf16(0.0))

    acc_ref[...] += jax.lax.dot_general(
        contrib_t.astype(bf16), oh, (((1,), (0,)), ((), ())),
        preferred_element_type=f32)

    @pl.when(i == nblocks - 1)
    def _fin():
        acc = acc_ref[...]
        cnt = acc[9:10, :]
        inv = jnp.where(cnt > 0, 1.0 / cnt, 0.0)
        out_ref[...] = acc * inv


def kernel(edge_emb, edge_index, distance_vec, lattice, batch, rbf,
           W1, W2, W_rbf, W_out):
    E, D = edge_emb.shape
    N = batch.shape[0]
    G = lattice.shape[0]
    DR = rbf.shape[1]
    B = _pick_block(E)
    nb = E // B

    idx0 = edge_index[0].reshape(nb, 1, B)
    batch2d = batch.reshape(N, 1)
    dvect = distance_vec.T

    in_specs = [
        pl.BlockSpec((B, D), lambda i: (i, 0)),
        pl.BlockSpec((B, DR), lambda i: (i, 0)),
        pl.BlockSpec((3, B), lambda i: (0, i)),
        pl.BlockSpec((1, 1, B), lambda i: (i, 0, 0)),
        pl.BlockSpec((N, 1), lambda i: (0, 0)),
        pl.BlockSpec((D, D), lambda i: (0, 0)),
        pl.BlockSpec((D, D), lambda i: (0, 0)),
        pl.BlockSpec((DR, D), lambda i: (0, 0)),
        pl.BlockSpec((D, 1), lambda i: (0, 0)),
    ]
    out = pl.pallas_call(
        functools.partial(_lattice_kernel, num_graphs=G, nblocks=nb,
                          num_nodes=N),
        grid=(nb,),
        in_specs=in_specs,
        out_specs=pl.BlockSpec((16, G), lambda i: (0, 0)),
        out_shape=jax.ShapeDtypeStruct((16, G), jnp.float32),
        scratch_shapes=[
            pltpu.VMEM((16, G), jnp.float32),
            pltpu.VMEM((1, G), jnp.int16),
            pltpu.VMEM((1, G), jnp.int16),
        ],
        compiler_params=pltpu.CompilerParams(
            dimension_semantics=("arbitrary",),
        ),
    )(edge_emb, rbf, dvect, idx0, batch2d, W1, W2, W_rbf, W_out)
    return jnp.transpose(out)[:, :9].reshape(G, 3, 3)
